# Initial kernel scaffold; baseline (speedup 1.0000x reference)
import jax
import jax.numpy as jnp
from jax.experimental import pallas as pl


def kernel(x, cur_pos, emb_table):
    raise NotImplementedError("write your pallas kernel here")


# speedup vs baseline: 1.0978x; 1.0978x over previous
import functools

import jax
import jax.numpy as jnp
from jax import lax
from jax.experimental import pallas as pl
from jax.experimental.pallas import tpu as pltpu
from jax.experimental.pallas import tpu_sc as plsc

EMBED = 128
T = 4
LANES = 16
CHUNKS = EMBED // LANES


def _poe(ctx, emb):
    i = jnp.arange(ctx, dtype=jnp.float32)[:, None]
    j = jnp.arange(emb)[None, :]
    even = (j % 2 == 0)
    exponent = jnp.where(even, j, j - 1).astype(jnp.float32) / emb
    ang = i / (10000.0 ** exponent)
    return jnp.where(even, jnp.sin(ang), jnp.cos(ang)).astype(jnp.float32)


def _rsqrt_newton(v):
    i = lax.bitcast_convert_type(v, jnp.int32)
    i = jnp.int32(0x5F3759DF) - lax.shift_right_logical(i, 1)
    y = lax.bitcast_convert_type(i, jnp.float32)
    for _ in range(3):
        y = y * (1.5 - 0.5 * v * y * y)
    return y


def _hsum(v):
    iota = lax.iota(jnp.int32, LANES)
    dnums = lax.GatherDimensionNumbers(
        offset_dims=(), collapsed_slice_dims=(0,), start_index_map=(0,))
    for k in (8, 4, 2, 1):
        idx = jnp.bitwise_xor(iota, k)
        v = v + lax.gather(v, idx[:, None], dimension_numbers=dnums,
                           slice_sizes=(1,),
                           mode=lax.GatherScatterMode.PROMISE_IN_BOUNDS)
    return v


def _make_sc_kernel(seq, num_cores, rows_per_w):
    mesh = plsc.VectorSubcoreMesh(core_axis_name="c", subcore_axis_name="s")

    @functools.partial(
        pl.kernel,
        mesh=mesh,
        out_type=jax.ShapeDtypeStruct((T, seq, EMBED), jnp.float32),
        scratch_types=[
            pltpu.VMEM((rows_per_w,), jnp.int32),
            pltpu.VMEM((rows_per_w, EMBED), jnp.float32),
            pltpu.VMEM((rows_per_w, EMBED), jnp.float32),
            pltpu.SemaphoreType.DMA,
        ],
    )
    def sc_kernel(x_hbm, table_hbm, poe_hbm, out_hbm, idx_v, rows_v, poe_v, sem):
        wid = lax.axis_index("s") * num_cores + lax.axis_index("c")
        base = wid * rows_per_w

        pltpu.sync_copy(x_hbm.at[pl.ds(base, rows_per_w)], idx_v)
        gather = pltpu.async_copy(table_hbm.at[idx_v], rows_v, sem)
        pltpu.sync_copy(poe_hbm.at[pl.ds(base, rows_per_w), :], poe_v)
        gather.wait()

        def row_body(r, carry):
            xs = []
            for j in range(CHUNKS):
                xj = rows_v[r, pl.ds(j * LANES, LANES)] + poe_v[r, pl.ds(j * LANES, LANES)]
                xs.append(xj)
            s = xs[0]
            q = xs[0] * xs[0]
            for j in range(1, CHUNKS):
                s = s + xs[j]
                q = q + xs[j] * xs[j]
            mean = _hsum(s) * (1.0 / EMBED)
            var = _hsum(q) * (1.0 / EMBED) - mean * mean
            inv = _rsqrt_newton(var + 1e-5)
            for j in range(CHUNKS):
                rows_v[r, pl.ds(j * LANES, LANES)] = (xs[j] - mean) * inv
            return carry

        lax.fori_loop(0, rows_per_w, row_body, 0)

        for t in range(T):
            pltpu.sync_copy(rows_v, out_hbm.at[t, pl.ds(base, rows_per_w), :])

    return sc_kernel


def kernel(x, cur_pos, emb_table):
    seq = x.shape[0]
    info = plsc.get_sparse_core_info()
    n_workers = info.num_cores * info.num_subcores
    rows_per_w = seq // n_workers
    poe = _poe(seq, EMBED)
    sc = _make_sc_kernel(seq, info.num_cores, rows_per_w)
    return sc(x, emb_table, poe)


# speedup vs baseline: 1.1606x; 1.0572x over previous
import functools

import jax
import jax.numpy as jnp
from jax import lax
from jax.experimental import pallas as pl
from jax.experimental.pallas import tpu as pltpu
from jax.experimental.pallas import tpu_sc as plsc

EMBED = 128
T = 4
LANES = 16
CHUNKS = EMBED // LANES


def _poe(ctx, emb):
    i = jnp.arange(ctx, dtype=jnp.float32)[:, None]
    j = jnp.arange(emb)[None, :]
    even = (j % 2 == 0)
    exponent = jnp.where(even, j, j - 1).astype(jnp.float32) / emb
    ang = i / (10000.0 ** exponent)
    return jnp.where(even, jnp.sin(ang), jnp.cos(ang)).astype(jnp.float32)


def _rsqrt_newton(v):
    i = lax.bitcast_convert_type(v, jnp.int32)
    i = jnp.int32(0x5F3759DF) - lax.shift_right_logical(i, 1)
    y = lax.bitcast_convert_type(i, jnp.float32)
    for _ in range(3):
        y = y * (1.5 - 0.5 * v * y * y)
    return y


def _hsum(v):
    iota = lax.iota(jnp.int32, LANES)
    dnums = lax.GatherDimensionNumbers(
        offset_dims=(), collapsed_slice_dims=(0,), start_index_map=(0,))
    for k in (8, 4, 2, 1):
        idx = jnp.bitwise_xor(iota, k)
        v = v + lax.gather(v, idx[:, None], dimension_numbers=dnums,
                           slice_sizes=(1,),
                           mode=lax.GatherScatterMode.PROMISE_IN_BOUNDS)
    return v


def _make_sc_kernel(seq, num_cores, rows_per_w):
    mesh = plsc.VectorSubcoreMesh(core_axis_name="c", subcore_axis_name="s")

    @functools.partial(
        pl.kernel,
        mesh=mesh,
        out_type=jax.ShapeDtypeStruct((T, seq, EMBED), jnp.float32),
        scratch_types=[
            pltpu.VMEM((rows_per_w,), jnp.int32),
            pltpu.VMEM((rows_per_w, EMBED), jnp.float32),
            pltpu.VMEM((rows_per_w, EMBED), jnp.float32),
            pltpu.SemaphoreType.DMA,
            pltpu.SemaphoreType.DMA,
            pltpu.SemaphoreType.DMA,
            pltpu.SemaphoreType.DMA,
        ],
    )
    def sc_kernel(x_hbm, table_hbm, poe_hbm, out_hbm,
                  idx_v, rows_v, poe_v, sem_a, sem_b, sem_p, sem_o):
        wid = lax.axis_index("s") * num_cores + lax.axis_index("c")
        base = wid * rows_per_w
        half = rows_per_w // 2
        chunk = 16
        n_chunks = rows_per_w // chunk

        pltpu.sync_copy(x_hbm.at[pl.ds(base, rows_per_w)], idx_v)
        g_a = pltpu.async_copy(table_hbm.at[idx_v.at[pl.ds(0, half)]],
                               rows_v.at[pl.ds(0, half), :], sem_a)
        g_b = pltpu.async_copy(table_hbm.at[idx_v.at[pl.ds(half, half)]],
                               rows_v.at[pl.ds(half, half), :], sem_b)
        g_p = pltpu.async_copy(poe_hbm.at[pl.ds(base, rows_per_w), :], poe_v, sem_p)

        def norm_two_rows(i, c0):
            outs = []
            for u in range(2):
                r = c0 + i * 2 + u
                xs = []
                for j in range(CHUNKS):
                    xj = (rows_v[r, pl.ds(j * LANES, LANES)]
                          + poe_v[r, pl.ds(j * LANES, LANES)])
                    xs.append(xj)
                s = xs[0]
                q = xs[0] * xs[0]
                for j in range(1, CHUNKS):
                    s = s + xs[j]
                    q = q + xs[j] * xs[j]
                outs.append((r, xs, s, q))
            for r, xs, s, q in outs:
                mean = _hsum(s) * (1.0 / EMBED)
                var = _hsum(q) * (1.0 / EMBED) - mean * mean
                inv = _rsqrt_newton(var + 1e-5)
                for j in range(CHUNKS):
                    rows_v[r, pl.ds(j * LANES, LANES)] = (xs[j] - mean) * inv
            return c0

        g_p.wait()
        out_copies = []
        for c in range(n_chunks):
            if c == 0:
                g_a.wait()
            if c == n_chunks // 2:
                g_b.wait()
            lax.fori_loop(0, chunk // 2, norm_two_rows, c * chunk)
            for t in range(T):
                out_copies.append(pltpu.async_copy(
                    rows_v.at[pl.ds(c * chunk, chunk), :],
                    out_hbm.at[t, pl.ds(base + c * chunk, chunk), :], sem_o))
        for d in out_copies:
            d.wait()

    return sc_kernel


def kernel(x, cur_pos, emb_table):
    seq = x.shape[0]
    info = plsc.get_sparse_core_info()
    n_workers = info.num_cores * info.num_subcores
    rows_per_w = seq // n_workers
    poe = _poe(seq, EMBED)
    sc = _make_sc_kernel(seq, info.num_cores, rows_per_w)
    return sc(x, emb_table, poe)


# speedup vs baseline: 1.2226x; 1.0534x over previous
import functools

import jax
import jax.numpy as jnp
import numpy as np
from jax import lax
from jax.experimental import pallas as pl
from jax.experimental.pallas import tpu as pltpu
from jax.experimental.pallas import tpu_sc as plsc

EMBED = 128
T = 4
LANES = 16
CHUNKS = EMBED // LANES


@functools.lru_cache(maxsize=None)
def _poe(ctx, emb):
    i = np.arange(ctx, dtype=np.float32)[:, None]
    j = np.arange(emb)[None, :]
    even = (j % 2 == 0)
    exponent = np.where(even, j, j - 1).astype(np.float32) / emb
    ang = i / (10000.0 ** exponent)
    return jnp.asarray(np.where(even, np.sin(ang), np.cos(ang)).astype(np.float32))


def _rsqrt_newton(v):
    i = lax.bitcast_convert_type(v, jnp.int32)
    i = jnp.int32(0x5F3759DF) - lax.shift_right_logical(i, 1)
    y = lax.bitcast_convert_type(i, jnp.float32)
    for _ in range(3):
        y = y * (1.5 - 0.5 * v * y * y)
    return y


def _hsum(v):
    iota = lax.iota(jnp.int32, LANES)
    dnums = lax.GatherDimensionNumbers(
        offset_dims=(), collapsed_slice_dims=(0,), start_index_map=(0,))
    for k in (8, 4, 2, 1):
        idx = jnp.bitwise_xor(iota, k)
        v = v + lax.gather(v, idx[:, None], dimension_numbers=dnums,
                           slice_sizes=(1,),
                           mode=lax.GatherScatterMode.PROMISE_IN_BOUNDS)
    return v


def _make_sc_kernel(seq, num_cores, rows_per_w):
    mesh = plsc.VectorSubcoreMesh(core_axis_name="c", subcore_axis_name="s")

    @functools.partial(
        pl.kernel,
        mesh=mesh,
        out_type=jax.ShapeDtypeStruct((T, seq, EMBED), jnp.float32),
        scratch_types=[
            pltpu.VMEM((rows_per_w,), jnp.int32),
            pltpu.VMEM((rows_per_w, EMBED), jnp.float32),
            pltpu.VMEM((rows_per_w, EMBED), jnp.float32),
            pltpu.SemaphoreType.DMA,
            pltpu.SemaphoreType.DMA,
            pltpu.SemaphoreType.DMA,
            pltpu.SemaphoreType.DMA,
        ],
    )
    def sc_kernel(x_hbm, table_hbm, poe_hbm, out_hbm,
                  idx_v, rows_v, poe_v, sem_a, sem_b, sem_p, sem_o):
        wid = lax.axis_index("s") * num_cores + lax.axis_index("c")
        base = wid * rows_per_w
        half = rows_per_w // 2
        chunk = 16
        n_chunks = rows_per_w // chunk

        pltpu.sync_copy(x_hbm.at[pl.ds(base, rows_per_w)], idx_v)
        g_a = pltpu.async_copy(table_hbm.at[idx_v.at[pl.ds(0, half)]],
                               rows_v.at[pl.ds(0, half), :], sem_a)
        g_b = pltpu.async_copy(table_hbm.at[idx_v.at[pl.ds(half, half)]],
                               rows_v.at[pl.ds(half, half), :], sem_b)
        g_p = pltpu.async_copy(poe_hbm.at[pl.ds(base, rows_per_w), :], poe_v, sem_p)

        def norm_two_rows(i, c0):
            outs = []
            for u in range(2):
                r = c0 + i * 2 + u
                xs = []
                for j in range(CHUNKS):
                    xj = (rows_v[r, pl.ds(j * LANES, LANES)]
                          + poe_v[r, pl.ds(j * LANES, LANES)])
                    xs.append(xj)
                s = xs[0]
                q = xs[0] * xs[0]
                for j in range(1, CHUNKS):
                    s = s + xs[j]
                    q = q + xs[j] * xs[j]
                outs.append((r, xs, s, q))
            for r, xs, s, q in outs:
                mean = _hsum(s) * (1.0 / EMBED)
                var = _hsum(q) * (1.0 / EMBED) - mean * mean
                inv = _rsqrt_newton(var + 1e-5)
                for j in range(CHUNKS):
                    rows_v[r, pl.ds(j * LANES, LANES)] = (xs[j] - mean) * inv
            return c0

        g_p.wait()
        out_copies = []
        for c in range(n_chunks):
            if c == 0:
                g_a.wait()
            if c == n_chunks // 2:
                g_b.wait()
            lax.fori_loop(0, chunk // 2, norm_two_rows, c * chunk)
            for t in range(T):
                out_copies.append(pltpu.async_copy(
                    rows_v.at[pl.ds(c * chunk, chunk), :],
                    out_hbm.at[t, pl.ds(base + c * chunk, chunk), :], sem_o))
        for d in out_copies:
            d.wait()

    return sc_kernel


def kernel(x, cur_pos, emb_table):
    seq = x.shape[0]
    info = plsc.get_sparse_core_info()
    n_workers = info.num_cores * info.num_subcores
    rows_per_w = seq // n_workers
    poe = _poe(seq, EMBED)
    sc = _make_sc_kernel(seq, info.num_cores, rows_per_w)
    return sc(x, emb_table, poe)
